# baseline (speedup 1.0000x reference)
import functools
import math

import jax
import jax.numpy as jnp
from jax import lax
from jax.experimental import pallas as pl
from jax.experimental.pallas import tpu as pltpu
from jax.experimental.pallas import tpu_sc as plsc

D = 64
SCALE = math.sqrt(D)
NC, NS, L = 2, 16, 16
NW = NC * NS
U = 128
B_ROWS = 4096
SEQ = 200
TOTAL = B_ROWS * SEQ
UNITS = TOTAL // U
UPW = UNITS // NW


@functools.partial(
    pl.kernel,
    mesh=plsc.VectorSubcoreMesh(core_axis_name="c", subcore_axis_name="s"),
    compiler_params=pltpu.CompilerParams(use_tc_tiling_on_sc=False),
    out_type=jax.ShapeDtypeStruct((TOTAL, D), jnp.float32),
    scratch_types=[
        pltpu.VMEM((UPW, U), jnp.int32),
        pltpu.VMEM((U, D), jnp.float32),
        pltpu.SemaphoreType.DMA,
    ],
)
def _emb_lookup(x_hbm, table_hbm, out_hbm, idx_v, rows_v, sem):
    c = lax.axis_index("c")
    s = lax.axis_index("s")
    wid = s * NC + c
    pltpu.sync_copy(x_hbm.at[pl.ds(wid * UPW, UPW)], idx_v)

    def unit(u, carry):
        pltpu.async_copy(table_hbm.at[idx_v.at[u]], rows_v, sem).wait()

        def mul_row(r, carry2):
            for j in range(D // L):
                sl = rows_v[r, pl.ds(j * L, L)]
                rows_v[r, pl.ds(j * L, L)] = sl * SCALE
            return carry2

        lax.fori_loop(0, U, mul_row, 0)
        pltpu.sync_copy(rows_v, out_hbm.at[pl.ds((wid * UPW + u) * U, U)])
        return carry

    lax.fori_loop(0, UPW, unit, 0)


def kernel(x, table):
    x2 = x.reshape(UNITS, U).astype(jnp.int32)
    out = _emb_lookup(x2, table)
    return out.reshape(B_ROWS, SEQ, D)
